# baseline (speedup 1.0000x reference)
import functools

import jax
import jax.numpy as jnp
from jax import lax
from jax.experimental import pallas as pl
from jax.experimental.pallas import tpu as pltpu
from jax.experimental.pallas import tpu_sc as plsc

NC = 2
NS = 16
NW = NC * NS
LANES = 16
CH = 32


def _sc_body(in_hbm, w_hbm, out_hbm, w_v, x_v):
    wid = lax.axis_index("s") * NC + lax.axis_index("c")
    batch, seq_len, dim = in_hbm.shape
    rows_per_w = seq_len // NW
    s_base = wid * rows_per_w
    vecs_per_row = dim // LANES

    def jloop(j, _):
        s0 = s_base + j * CH
        pltpu.sync_copy(w_hbm.at[pl.ds(s0, CH), :], w_v)

        def bloop(b, _):
            pltpu.sync_copy(in_hbm.at[b, pl.ds(s0, CH), :], x_v)

            def rloop(r, _):
                for c in range(vecs_per_row):
                    sl = pl.ds(c * LANES, LANES)
                    x_v[r, sl] = x_v[r, sl] + w_v[r, sl]
                return 0

            lax.fori_loop(0, CH, rloop, 0)
            pltpu.sync_copy(x_v, out_hbm.at[b, pl.ds(s0, CH), :])
            return 0

        lax.fori_loop(0, batch, bloop, 0)
        return 0

    lax.fori_loop(0, rows_per_w // CH, jloop, 0)


def kernel(inputs, W):
    batch, seq_len, dim = inputs.shape
    mesh = plsc.VectorSubcoreMesh(core_axis_name="c", subcore_axis_name="s")
    k = functools.partial(
        pl.kernel,
        out_type=jax.ShapeDtypeStruct((batch, seq_len, dim), inputs.dtype),
        mesh=mesh,
        scratch_types=[
            pltpu.VMEM((CH, dim), jnp.float32),
            pltpu.VMEM((CH, dim), jnp.float32),
        ],
    )(_sc_body)
    return k(inputs, W)
